# baseline (speedup 1.0000x reference)
import jax
import jax.numpy as jnp
import numpy as np
from jax import lax
from jax.experimental import pallas as pl
from jax.experimental.pallas import tpu as pltpu
from jax.experimental.pallas import tpu_sc as plsc

_B, _L, _D, _H, _KMAX = 2, 2048, 1024, 16, 32
_NROWS = _B * _L
_RA = 1024
_RB = 1024
_NW = 32
_RPW = _NROWS // _NW
_NWORDS = _L // 32

_M_PROJ = (
    (0.8130764, -1.2237617), (-0.3871328, 0.6545847),
    (1.7382764, 0.2871933), (-0.9170771, -1.8237641),
    (0.2948277, 1.1349734), (-1.4862293, 0.4456314),
    (0.6198434, -0.7381172), (1.0737158, 1.6233476),
    (-0.5632871, -0.1987243), (0.9213477, -1.3376218),
    (-1.1098764, 0.8361299), (0.3748293, -0.4472917),
    (1.2987364, 1.0038472), (-0.7364871, 1.4458261),
    (0.1847293, -0.9983174), (-1.6523781, 0.5578213),
)


_RQ = 512


def _fused_body(q_ref, k_ref, w_ref, mproj_ref, p_ref, words_ref, cnt_ref,
                sk0, sk1):
    b = pl.program_id(0)
    p = pl.program_id(1)
    wtm = lax.dot_general(w_ref[...], mproj_ref[...],
                          (((0,), (0,)), ((), ())),
                          preferred_element_type=jnp.float32)

    def fp2(x):
        xb = (x > 0).astype(jnp.float32)
        return jnp.dot(xb, wtm, preferred_element_type=jnp.float32)

    @pl.when(p < 4)
    def _():
        gk = fp2(k_ref[...].reshape(_RQ, _D))
        sk0[pl.ds(p * (_RQ // 128), _RQ // 128), :] = (
            gk[:, 0].reshape(_RQ // 128, 128))
        sk1[pl.ds(p * (_RQ // 128), _RQ // 128), :] = (
            gk[:, 1].reshape(_RQ // 128, 128))

    @pl.when(p >= 4)
    def _():
        fq = fp2(q_ref[...].reshape(_RQ, _D))
        q0 = fq[:, 0:1]
        q1 = fq[:, 1:2]
        k0 = sk0[...].reshape(1, _L)
        k1 = sk1[...].reshape(1, _L)
        m = (q0 == k0) & (q1 == k1)
        mb = m.astype(jnp.bfloat16)
        acc = jnp.dot(mb, p_ref[...], preferred_element_type=jnp.float32)
        lo = acc[:, :_NWORDS].astype(jnp.int32)
        hi = acc[:, _NWORDS:2 * _NWORDS].astype(jnp.int32)
        words_ref[...] = lo | (hi << 16)
        cnt_ref[...] = acc[:, 2 * _NWORDS].astype(jnp.int32).reshape(
            cnt_ref.shape)


def _sc_extract_body(words_hbm, cnt_hbm, out_hbm, cnt_v, words_v, out_v,
                     sem):
    wid = lax.axis_index("c") * 16 + lax.axis_index("s")
    base = wid * _RPW
    cnt_cp = pltpu.async_copy(cnt_hbm.at[pl.ds(base, _RPW)], cnt_v, sem)

    neg1 = jnp.full((16,), -1, jnp.int32)
    lane0 = lax.iota(jnp.int32, 16) < 1

    def init16(i, z):
        out_v[pl.ds(i * 16, 16)] = neg1
        return z

    lax.fori_loop(0, _RPW * _KMAX // 16, init16, 0, unroll=8)
    cnt_cp.wait()

    acc = jnp.zeros((16,), jnp.int32)
    for i in range(_RPW // 16):
        acc = acc | cnt_v[pl.ds(i * 16, 16)]
    tile_has = plsc.all_reduce_population_count(acc != 0)[0]

    @pl.when(tile_has > 0)
    def _():
        pltpu.sync_copy(words_hbm.at[pl.ds(base * _NWORDS, _RPW * _NWORDS)],
                        words_v.at[pl.ds(0, _RPW * _NWORDS)])

        def do_row(r, z):
            rb = r * _NWORDS
            rcnt = cnt_v[pl.ds(r & ~15, 16)]
            has = plsc.all_reduce_population_count(
                (rcnt != 0) & (lax.iota(jnp.int32, 16) == (r & 15)))[0]

            @pl.when(has > 0)
            def _():
                def word_loop(w, got):
                    word = words_v[pl.ds(rb + w, 16)][0]

                    def bit_loop(p, got2, word=word):
                        take = ((((word >> p) & 1) > 0) & (got2 < _KMAX))

                        @pl.when(take)
                        def _(w=w, p=p, got2=got2):
                            idx = jnp.full((16,), r * _KMAX + got2,
                                           jnp.int32)
                            pos = jnp.full((16,), w * 32 + p, jnp.int32)
                            plsc.store_scatter(out_v, [idx], pos,
                                               mask=lane0)

                        return got2 + take.astype(jnp.int32)

                    return lax.cond(
                        word != 0,
                        lambda go, word=word, w=w:
                            lax.fori_loop(0, 32, bit_loop, go),
                        lambda go: go, got)

                lax.fori_loop(0, _NWORDS, word_loop, jnp.int32(0))

            return z

        lax.fori_loop(0, _RPW, do_row, 0)

    pltpu.sync_copy(out_v, out_hbm.at[pl.ds(base * _KMAX, _RPW * _KMAX)])


def _build_pack_matrix():
    j = np.arange(_L)
    w = j // 32
    t = j % 32
    col = np.where(t < 16, w, _NWORDS + w)
    val = (1 << (t % 16)).astype(np.float32)
    p = (col[:, None] == np.arange(2 * _NWORDS + 16)[None, :]) * val[:, None]
    p[:, 2 * _NWORDS] = 1.0
    return jnp.asarray(p, jnp.bfloat16)


def kernel(query, key, head_idx, W, b):
    del head_idx
    q4 = query.reshape(_B, 2, _RB, _D)
    mproj = jnp.asarray(_M_PROJ, jnp.float32)
    pmat = _build_pack_matrix()

    q4 = query.reshape(_B, 4, _RQ, _D)
    k4 = key.reshape(_B, 4, _RQ, _D)
    words, cnt = pl.pallas_call(
        _fused_body,
        grid=(_B, 8),
        in_specs=[
            pl.BlockSpec((1, 1, _RQ, _D),
                         lambda bi, p: (bi, jnp.maximum(p - 4, 0), 0, 0)),
            pl.BlockSpec((1, 1, _RQ, _D),
                         lambda bi, p: (bi, jnp.minimum(p, 3), 0, 0)),
            pl.BlockSpec((_H, _D), lambda bi, p: (0, 0)),
            pl.BlockSpec((_H, 2), lambda bi, p: (0, 0)),
            pl.BlockSpec((_L, 2 * _NWORDS + 16), lambda bi, p: (0, 0)),
        ],
        out_specs=[
            pl.BlockSpec((_RQ, _NWORDS),
                         lambda bi, p: (bi * 4 + jnp.maximum(p - 4, 0), 0)),
            pl.BlockSpec((1, _RQ // 128, 128),
                         lambda bi, p: (bi * 4 + jnp.maximum(p - 4, 0), 0, 0)),
        ],
        out_shape=[
            jax.ShapeDtypeStruct((_NROWS, _NWORDS), jnp.int32),
            jax.ShapeDtypeStruct((_NROWS // _RQ, _RQ // 128, 128),
                                 jnp.int32),
        ],
        scratch_shapes=[
            pltpu.VMEM((_L // 128, 128), jnp.float32),
            pltpu.VMEM((_L // 128, 128), jnp.float32),
        ],
    )(q4, k4, W, mproj, pmat)

    out = _sc_first_k(words.reshape(_NROWS * _NWORDS), cnt.reshape(_NROWS))
    return out.reshape(_B, _L, _KMAX)


def _sc_first_k(words_flat, cnt_flat):
    return pl.kernel(
        _sc_extract_body,
        out_type=jax.ShapeDtypeStruct((_NROWS * _KMAX,), jnp.int32),
        mesh=plsc.VectorSubcoreMesh(core_axis_name="c", subcore_axis_name="s",
                                    num_cores=2, num_subcores=16),
        compiler_params=pltpu.CompilerParams(needs_layout_passes=False),
        scratch_types=[
            pltpu.VMEM((_RPW,), jnp.int32),
            pltpu.VMEM((_RPW * _NWORDS + 16,), jnp.int32),
            pltpu.VMEM((_RPW * _KMAX,), jnp.int32),
            pltpu.SemaphoreType.DMA,
        ],
    )(words_flat, cnt_flat)


# speedup vs baseline: 1.1708x; 1.1708x over previous
import jax
import jax.numpy as jnp
import numpy as np
from jax import lax
from jax.experimental import pallas as pl
from jax.experimental.pallas import tpu as pltpu
from jax.experimental.pallas import tpu_sc as plsc

_B, _L, _D, _H, _KMAX = 2, 2048, 1024, 16, 32
_NROWS = _B * _L
_RA = 1024
_RB = 1024
_NW = 32
_RPW = _NROWS // _NW
_NWORDS = _L // 32

_M_PROJ = (
    (0.8130764, -1.2237617), (-0.3871328, 0.6545847),
    (1.7382764, 0.2871933), (-0.9170771, -1.8237641),
    (0.2948277, 1.1349734), (-1.4862293, 0.4456314),
    (0.6198434, -0.7381172), (1.0737158, 1.6233476),
    (-0.5632871, -0.1987243), (0.9213477, -1.3376218),
    (-1.1098764, 0.8361299), (0.3748293, -0.4472917),
    (1.2987364, 1.0038472), (-0.7364871, 1.4458261),
    (0.1847293, -0.9983174), (-1.6523781, 0.5578213),
)


def _fused_body(q_ref, k_ref, w_ref, mproj_ref, p_ref, words_ref, cnt_ref):
    wtm = lax.dot_general(w_ref[...], mproj_ref[...],
                          (((0,), (0,)), ((), ())),
                          preferred_element_type=jnp.float32)

    def fp2(x):
        xb = (x > 0).astype(jnp.float32)
        return jnp.dot(xb, wtm, preferred_element_type=jnp.float32)

    gk = fp2(k_ref[...].reshape(_L, _D))
    k0 = gk[:, 0].reshape(1, _L)
    k1 = gk[:, 1].reshape(1, _L)
    fq = fp2(q_ref[...].reshape(_L, _D))

    p = p_ref[...]
    for h in range(_L // _RB):
        q0 = fq[h * _RB:(h + 1) * _RB, 0:1]
        q1 = fq[h * _RB:(h + 1) * _RB, 1:2]
        m = (q0 == k0) & (q1 == k1)
        mb = m.astype(jnp.bfloat16)
        acc = jnp.dot(mb, p, preferred_element_type=jnp.float32)
        lo = acc[:, :_NWORDS].astype(jnp.int32)
        hi = acc[:, 128:128 + _NWORDS].astype(jnp.int32)
        words_ref[pl.ds(h * _RB, _RB), :] = lo | (hi << 16)
        cnt_ref[pl.ds(h * (_RB // 128), _RB // 128), :] = (
            acc[:, 256].astype(jnp.int32).reshape(_RB // 128, 128))


def _sc_extract_body(words_hbm, cnt_hbm, out_hbm, cnt_v, words_v, out_v,
                     sem):
    wid = lax.axis_index("c") * 16 + lax.axis_index("s")
    base = wid * _RPW
    cnt_cp = pltpu.async_copy(cnt_hbm.at[pl.ds(base, _RPW)], cnt_v, sem)

    neg1 = jnp.full((16,), -1, jnp.int32)
    lane0 = lax.iota(jnp.int32, 16) < 1

    def init16(i, z):
        out_v[pl.ds(i * 16, 16)] = neg1
        return z

    lax.fori_loop(0, _RPW * _KMAX // 16, init16, 0, unroll=8)
    cnt_cp.wait()

    acc = jnp.zeros((16,), jnp.int32)
    for i in range(_RPW // 16):
        acc = acc | cnt_v[pl.ds(i * 16, 16)]
    tile_has = plsc.all_reduce_population_count(acc != 0)[0]

    @pl.when(tile_has > 0)
    def _():
        pltpu.sync_copy(words_hbm.at[pl.ds(base * _NWORDS, _RPW * _NWORDS)],
                        words_v)
        lanes = lax.iota(jnp.int32, 16)

        def do_row(r, z):
            crow = cnt_v[pl.ds(pl.multiple_of(r & ~15, 16), 16)]
            has = plsc.all_reduce_population_count(
                (crow != 0) & (lanes == (r & 15)))[0]

            @pl.when(has > 0)
            def _():
                rb = r * _NWORDS

                def group(g, got):
                    wv = words_v[pl.ds(pl.multiple_of(rb + g * 16, 16), 16)]
                    gpop = plsc.all_reduce_population_count(wv != 0)[0]

                    def dense(got2):
                        for lane in range(16):
                            word = wv[lane]
                            wbase = (g * 16 + lane) * 32

                            def bit_loop(p, got3, word=word, wbase=wbase):
                                take = ((((word >> p) & 1) > 0)
                                        & (got3 < _KMAX))

                                @pl.when(take)
                                def _():
                                    idx = jnp.full((16,), r * _KMAX + got3,
                                                   jnp.int32)
                                    pos = jnp.full((16,), wbase + p,
                                                   jnp.int32)
                                    plsc.store_scatter(out_v, [idx], pos,
                                                       mask=lane0)

                                return got3 + take.astype(jnp.int32)

                            got2 = lax.cond(
                                word != 0,
                                lambda go, word=word:
                                    lax.fori_loop(0, 32, bit_loop, go),
                                lambda go: go, got2)
                        return got2

                    return lax.cond(gpop > 0, dense, lambda go: go, got)

                lax.fori_loop(0, 4, group, jnp.int32(0))

            return z

        lax.fori_loop(0, _RPW, do_row, 0)

    pltpu.sync_copy(out_v, out_hbm.at[pl.ds(base * _KMAX, _RPW * _KMAX)])


def _build_pack_matrix():
    j = np.arange(_L)
    w = j // 32
    t = j % 32
    col = np.where(t < 16, w, 128 + w)
    val = (1 << (t % 16)).astype(np.float32)
    p = (col[:, None] == np.arange(272)[None, :]) * val[:, None]
    p[:, 256] = 1.0
    return jnp.asarray(p, jnp.bfloat16)


def kernel(query, key, head_idx, W, b):
    del head_idx
    q4 = query.reshape(_B, 2, _RB, _D)
    mproj = jnp.asarray(_M_PROJ, jnp.float32)
    pmat = _build_pack_matrix()

    words, cnt = pl.pallas_call(
        _fused_body,
        grid=(_B,),
        in_specs=[
            pl.BlockSpec((1, _L, _D), lambda bi: (bi, 0, 0)),
            pl.BlockSpec((1, _L, _D), lambda bi: (bi, 0, 0)),
            pl.BlockSpec((_H, _D), lambda bi: (0, 0)),
            pl.BlockSpec((_H, 2), lambda bi: (0, 0)),
            pl.BlockSpec((_L, 272), lambda bi: (0, 0)),
        ],
        out_specs=[
            pl.BlockSpec((_L, _NWORDS), lambda bi: (bi, 0)),
            pl.BlockSpec((_L // 128, 128), lambda bi: (bi, 0)),
        ],
        out_shape=[
            jax.ShapeDtypeStruct((_NROWS, _NWORDS), jnp.int32),
            jax.ShapeDtypeStruct((_NROWS // 128, 128), jnp.int32),
        ],
    )(query, key, W, mproj, pmat)

    out = _sc_first_k(words.reshape(_NROWS * _NWORDS), cnt.reshape(_NROWS))
    return out.reshape(_B, _L, _KMAX)


def _sc_first_k(words_flat, cnt_flat):
    return pl.kernel(
        _sc_extract_body,
        out_type=jax.ShapeDtypeStruct((_NROWS * _KMAX,), jnp.int32),
        mesh=plsc.VectorSubcoreMesh(core_axis_name="c", subcore_axis_name="s",
                                    num_cores=2, num_subcores=16),
        compiler_params=pltpu.CompilerParams(needs_layout_passes=False),
        scratch_types=[
            pltpu.VMEM((_RPW,), jnp.int32),
            pltpu.VMEM((_RPW * _NWORDS,), jnp.int32),
            pltpu.VMEM((_RPW * _KMAX,), jnp.int32),
            pltpu.SemaphoreType.DMA,
        ],
    )(words_flat, cnt_flat)
